# Initial kernel scaffold; baseline (speedup 1.0000x reference)
import jax
import jax.numpy as jnp
from jax.experimental import pallas as pl


def kernel(x, edge_index, edge_weight, enc0_zr_W, enc0_zr_b, enc0_h_W, enc0_h_b, enc1_zr_W, enc1_zr_b, enc1_h_W, enc1_h_b, dec0_zr_W, dec0_zr_b, dec0_h_W, dec0_h_b, dec1_zr_W, dec1_zr_b, dec1_h_W, dec1_h_b, dec_lin_W, dec_lin_b):
    raise NotImplementedError("write your pallas kernel here")


# speedup vs baseline: 1.0000x; 1.0000x over previous
import jax
import jax.numpy as jnp
from jax.experimental import pallas as pl

_N = 10000
_KD = 2
_NL = 2
_TIN = 12
_TOUT = 12


def _pp(edge_index, ef):
    src, dst = edge_index[0], edge_index[1]
    src_all = jnp.concatenate([src, dst + _N])
    dst_all = jnp.concatenate([dst, src + _N])
    ef_all = jnp.concatenate([ef, ef])
    deg = jax.ops.segment_sum(ef_all, dst_all, num_segments=2 * _N)
    d_inv = 1.0 / jnp.clip(deg, 1.0)
    return src_all, dst_all, d_inv[src_all] * ef_all


def _pr(h, src, dst, ef):
    m = h[src] * ef[:, None, None]
    return jax.ops.segment_sum(m, dst, num_segments=2 * _N)


def _dc(feat, W, b, src, dst, ef):
    outs = [feat]
    x0 = jnp.concatenate([feat, feat], axis=0)
    x1 = _pr(x0, src, dst, ef)
    outs.append(x1[:_N]); outs.append(x1[_N:])
    for _ in range(2, _KD + 1):
        x2 = 2.0 * _pr(x1, src, dst, ef) - x0
        outs.append(x2[:_N]); outs.append(x2[_N:])
        x1, x0 = x2, x1
    return jnp.concatenate(outs, axis=2) @ W + b


def _gru(feat, state, zr_W, zr_b, h_W, h_b, src, dst, ef):
    inp = jnp.concatenate([feat, state], axis=-1)
    zr = jax.nn.sigmoid(_dc(inp, zr_W, zr_b, src, dst, ef))
    r, u = jnp.split(zr, 2, axis=-1)
    c = jnp.tanh(_dc(jnp.concatenate([feat, r * state], axis=-1), h_W, h_b, src, dst, ef))
    return u * state + (1.0 - u) * c


def _step(feat, states, params, src, dst, ef):
    prev = feat
    ns = []
    for l in range(_NL):
        prev = _gru(prev, states[l], *params[l], src, dst, ef)
        ns.append(prev)
    return prev, ns


def kernel(x, edge_index, edge_weight, enc0_zr_W, enc0_zr_b, enc0_h_W, enc0_h_b, enc1_zr_W, enc1_zr_b, enc1_h_W, enc1_h_b, dec0_zr_W, dec0_zr_b, dec0_h_W, dec0_h_b, dec1_zr_W, dec1_zr_b, dec1_h_W, dec1_h_b, dec_lin_W, dec_lin_b):
    enc = [(enc0_zr_W, enc0_zr_b, enc0_h_W, enc0_h_b), (enc1_zr_W, enc1_zr_b, enc1_h_W, enc1_h_b)]
    dec = [(dec0_zr_W, dec0_zr_b, dec0_h_W, dec0_h_b), (dec1_zr_W, dec1_zr_b, dec1_h_W, dec1_h_b)]
    src, dst, ef = _pp(edge_index, edge_weight)
    nfeat = jnp.transpose(x, (1, 2, 0, 3))
    B = x.shape[0]
    states = [jnp.zeros((_N, B, 64), jnp.float32) for _ in range(_NL)]
    for t in range(_TIN):
        _, states = _step(nfeat[t], states, enc, src, dst, ef)
    layer_in = jnp.zeros((_N, B, 1), jnp.float32)
    outs = []
    for t in range(_TOUT):
        prev, states = _step(layer_in, states, dec, src, dst, ef)
        layer_out = prev @ dec_lin_W + dec_lin_b
        outs.append(layer_out)
        layer_in = layer_out
    out = jnp.stack(outs, axis=1)[..., 0]
    return jnp.transpose(out, (2, 1, 0))


# speedup vs baseline: 1.4004x; 1.4004x over previous
import functools

import jax
import jax.numpy as jnp
from jax import lax
from jax.experimental import pallas as pl
from jax.experimental.pallas import tpu as pltpu
from jax.experimental.pallas import tpu_sc as plsc

N = 10000
N2 = 20000
E2 = 320000
NW = 32
NC = 2
RPW = 640
N2P = NW * RPW
CHK = 128
NJ = 4
SUP = NJ * CHK
LE = E2 + NW * SUP


def _make_prop(C, R):
  mesh = plsc.VectorSubcoreMesh(core_axis_name="c", subcore_axis_name="s",
                                num_cores=NC, num_subcores=16)

  @functools.partial(
      pl.kernel,
      out_type=jax.ShapeDtypeStruct((N2P * C,), jnp.float32),
      mesh=mesh,
      scratch_types=[
          pltpu.VMEM((SUP,), jnp.int32),
          pltpu.VMEM((SUP,), jnp.float32),
          pltpu.VMEM((SUP,), jnp.int32),
          pltpu.VMEM((2, CHK, C), jnp.float32),
          pltpu.VMEM((RPW * C,), jnp.float32),
          pltpu.VMEM((2 * NW * 16,), jnp.int32),
          pltpu.SemaphoreType.DMA,
          pltpu.SemaphoreType.DMA,
          pltpu.SemaphoreType.DMA,
          pltpu.SemaphoreType.DMA,
          pltpu.SemaphoreType.DMA,
      ],
  )
  def prop(tbl, idxh, wh, dlh, meta, out,
           idx_st, w_st, dl_st, rows, acc, meta_v, g0, g1, s0, s1, s2):
    wid = lax.axis_index("s") * NC + lax.axis_index("c")
    pltpu.sync_copy(meta, meta_v)
    soff = meta_v[pl.ds(wid * 16, 16)][0]
    nsc = meta_v[pl.ds((NW + wid) * 16, 16)][0]

    def zrow(r, _):
      acc[pl.ds(r * 16, 16)] = jnp.zeros((16,), jnp.float32)
      return 0
    lax.fori_loop(0, RPW * C // 16, zrow, 0)

    gsem = [g0, g1]

    def sup_body(s, _):
      base = pl.multiple_of(soff + s * SUP, SUP)
      d1 = pltpu.async_copy(idxh.at[pl.ds(base, SUP)], idx_st, s0)
      d2 = pltpu.async_copy(wh.at[pl.ds(base, SUP)], w_st, s1)
      d3 = pltpu.async_copy(dlh.at[pl.ds(base, SUP)], dl_st, s2)
      d1.wait(); d2.wait(); d3.wait()
      descs = [None, None]
      descs[0] = pltpu.async_copy(tbl.at[idx_st.at[pl.ds(0, CHK)]],
                                  rows.at[0], g0)
      for j in range(NJ):
        b = j % 2
        descs[b].wait()
        if j < NJ - 1:
          nb = (j + 1) % 2
          descs[nb] = pltpu.async_copy(
              tbl.at[idx_st.at[pl.ds((j + 1) * CHK, CHK)]], rows.at[nb],
              gsem[nb])

        def blk_body(blk, _, j=j, b=b):
          e0 = j * CHK + blk * 16
          dlv = dl_st[pl.ds(e0, 16)]
          wv = w_st[pl.ds(e0, 16)]
          for e in range(16):
            dl_s = dlv[e]
            w_s = wv[e]
            rbase = dl_s * C
            for c in range(C // 16):
              vals = rows[b, blk * 16 + e, pl.ds(c * 16, 16)] * w_s
              plsc.addupdate(acc.at[pl.ds(rbase + c * 16, 16)], vals)
          return 0
        lax.fori_loop(0, CHK // 16, blk_body, 0)
      return 0
    lax.fori_loop(0, nsc, sup_body, 0)
    pltpu.sync_copy(acc, out.at[pl.ds(wid * (RPW * C), RPW * C)])

  return prop


_PROP_CACHE = {}


def _get_prop(C, R):
  if (C, R) not in _PROP_CACHE:
    _PROP_CACHE[(C, R)] = jax.jit(_make_prop(C, R))
  return _PROP_CACHE[(C, R)]


def _preprocess(edge_index, edge_weight):
  src, dst = edge_index[0], edge_index[1]
  idx1 = jnp.concatenate([src, dst])
  idx2 = jnp.concatenate([src, dst + N])
  dstf = jnp.concatenate([dst, src + N])
  srcf = idx2
  ef = jnp.concatenate([edge_weight, edge_weight])
  deg = jax.ops.segment_sum(ef, dstf, num_segments=N2)
  w = ef * (1.0 / jnp.clip(deg, 1.0))[srcf]
  order = jnp.argsort(dstf)
  ds = dstf[order]
  wid_e = ds // RPW
  dl = ds - wid_e * RPW
  cnt = jnp.bincount(wid_e, length=NW).astype(jnp.int32)
  padded = ((cnt + SUP - 1) // SUP) * SUP
  off = jnp.concatenate([jnp.zeros((1,), jnp.int32), jnp.cumsum(padded)[:-1]])
  prefix = jnp.concatenate([jnp.zeros((1,), jnp.int32), jnp.cumsum(cnt)[:-1]])
  pos = off[wid_e] + jnp.arange(E2, dtype=jnp.int32) - prefix[wid_e]
  i1L = jnp.zeros((LE,), jnp.int32).at[pos].set(idx1[order])
  i2L = jnp.zeros((LE,), jnp.int32).at[pos].set(idx2[order])
  wL = jnp.zeros((LE,), jnp.float32).at[pos].set(w[order])
  dlL = jnp.zeros((LE,), jnp.int32).at[pos].set(dl)
  meta = jnp.concatenate([jnp.repeat(off.astype(jnp.int32), 16),
                          jnp.repeat((padded // SUP).astype(jnp.int32), 16)])
  return dict(i1=i1L, i2=i2L, w=wL, dl=dlL, meta=meta)


BR = 400


def _make_conv_tc(P, F, kind):
  GB = N // BR
  fout = 128 if kind == "zr" else 64
  nspec = pl.BlockSpec((BR, P), lambda i: (i, 0))
  aspec = pl.BlockSpec((BR, P), lambda i: (i, 0))
  bspec = pl.BlockSpec((BR, P), lambda i: (i + GB, 0))
  wspec = pl.BlockSpec((5, P, fout), lambda i: (0, 0, 0))
  bbspec = pl.BlockSpec((1, fout), lambda i: (0, 0))
  sspec = pl.BlockSpec((BR, 64), lambda i: (i, 0))

  def acc5(inp, x1a, x1b, x2a, x2b, W, b):
    m0 = inp[...]
    return (m0 @ W[0] + x1a[...] @ W[1] + x1b[...] @ W[2]
            + (2.0 * x2a[...] - m0) @ W[3] + (2.0 * x2b[...] - m0) @ W[4]
            + b[...])

  if kind == "zr":
    def body(inp, x1a, x1b, x2a, x2b, W, b, o_inp2, o_u):
      zr = jax.nn.sigmoid(acc5(inp, x1a, x1b, x2a, x2b, W, b))
      r, u = zr[:, :64], zr[:, 64:]
      m0 = inp[...]
      pieces = [m0[:, :F], r * m0[:, F:F + 64]]
      if P - F - 64:
        pieces.append(jnp.zeros((BR, P - F - 64), jnp.float32))
      o_inp2[...] = jnp.concatenate(pieces, axis=1)
      o_u[...] = u
    return pl.pallas_call(
        body, grid=(GB,),
        in_specs=[nspec, aspec, bspec, aspec, bspec, wspec, bbspec],
        out_specs=[nspec, sspec],
        out_shape=[jax.ShapeDtypeStruct((N, P), jnp.float32),
                   jax.ShapeDtypeStruct((N, 64), jnp.float32)])

  if kind == "h":
    def body(inp2, x1a, x1b, x2a, x2b, W, b, u, st, o_h):
      c = jnp.tanh(acc5(inp2, x1a, x1b, x2a, x2b, W, b))
      o_h[...] = u[...] * st[...] + (1.0 - u[...]) * c
    return pl.pallas_call(
        body, grid=(GB,),
        in_specs=[nspec, aspec, bspec, aspec, bspec, wspec, bbspec,
                  sspec, sspec],
        out_specs=sspec,
        out_shape=jax.ShapeDtypeStruct((N, 64), jnp.float32))

  def body(inp2, x1a, x1b, x2a, x2b, W, b, u, st, Wd, bd, o_h, o_y):
    c = jnp.tanh(acc5(inp2, x1a, x1b, x2a, x2b, W, b))
    h = u[...] * st[...] + (1.0 - u[...]) * c
    o_h[...] = h
    o_y[...] = h @ Wd[...] + bd[...]
  return pl.pallas_call(
      body, grid=(GB,),
      in_specs=[nspec, aspec, bspec, aspec, bspec, wspec, bbspec,
                sspec, sspec,
                pl.BlockSpec((64, 8), lambda i: (0, 0)),
                pl.BlockSpec((1, 8), lambda i: (0, 0))],
      out_specs=[sspec, pl.BlockSpec((BR, 8), lambda i: (i, 0))],
      out_shape=[jax.ShapeDtypeStruct((N, 64), jnp.float32),
                 jax.ShapeDtypeStruct((N, 8), jnp.float32)])


_TC_CACHE = {}


def _get_tc(kind, P, F):
  if (kind, P, F) not in _TC_CACHE:
    _TC_CACHE[(kind, P, F)] = jax.jit(_make_conv_tc(P, F, kind))
  return _TC_CACHE[(kind, P, F)]


def _pad_w(W, cin, P, fout):
  Wr = W.reshape(5, cin, fout)
  return jnp.pad(Wr, ((0, 0), (0, P - cin), (0, 0)))


def _props(inp_p, C, ew):
  pk1 = _get_prop(C, N)
  pk2 = _get_prop(C, N2)
  x1 = pk1(inp_p, ew["i1"], ew["w"], ew["dl"], ew["meta"]).reshape(N2P, C)
  x2r = pk2(x1, ew["i2"], ew["w"], ew["dl"],
            ew["meta"]).reshape(N2P, C)
  return x1, x2r


def _gru(feat, stt, WzrP, bzr, WhP, bh, F, P, ew, dec_lin=None):
  pad = P - (F + 64)
  inp = jnp.pad(jnp.concatenate([feat, stt], axis=1), ((0, 0), (0, pad)))
  x1, x2 = _props(inp, P, ew)
  inp2, u = _get_tc("zr", P, F)(inp, x1, x1, x2, x2, WzrP, bzr)
  y1, y2 = _props(inp2, P, ew)
  if dec_lin is None:
    h = _get_tc("h", P, F)(inp2, y1, y1, y2, y2, WhP, bh, u, stt)
    return h, None
  h, y = _get_tc("hd", P, F)(inp2, y1, y1, y2, y2, WhP, bh, u, stt,
                             dec_lin[0], dec_lin[1])
  return h, y


def kernel(x, edge_index, edge_weight, enc0_zr_W, enc0_zr_b, enc0_h_W,
           enc0_h_b, enc1_zr_W, enc1_zr_b, enc1_h_W, enc1_h_b, dec0_zr_W,
           dec0_zr_b, dec0_h_W, dec0_h_b, dec1_zr_W, dec1_zr_b, dec1_h_W,
           dec1_h_b, dec_lin_W, dec_lin_b):
  ew = _preprocess(edge_index, edge_weight)
  enc_dims = [(2, 66, 128), (64, 128, 128)]
  dec_dims = [(1, 65, 128), (64, 128, 128)]
  enc = [(_pad_w(enc0_zr_W, 66, 128, 128), enc0_zr_b,
          _pad_w(enc0_h_W, 66, 128, 64), enc0_h_b),
         (_pad_w(enc1_zr_W, 128, 128, 128), enc1_zr_b,
          _pad_w(enc1_h_W, 128, 128, 64), enc1_h_b)]
  dec = [(_pad_w(dec0_zr_W, 65, 128, 128), dec0_zr_b,
          _pad_w(dec0_h_W, 65, 128, 64), dec0_h_b),
         (_pad_w(dec1_zr_W, 128, 128, 128), dec1_zr_b,
          _pad_w(dec1_h_W, 128, 128, 64), dec1_h_b)]

  enc = [tuple(w if w.ndim != 1 else w.reshape(1, -1) for w in ws)
         for ws in enc]
  dec = [tuple(w if w.ndim != 1 else w.reshape(1, -1) for w in ws)
         for ws in dec]
  WdP = jnp.pad(dec_lin_W, ((0, 0), (0, 7)))
  bdP = jnp.pad(dec_lin_b.reshape(1, 1), ((0, 0), (0, 7)))

  xs = x[0]
  states = [jnp.zeros((N, 64), jnp.float32) for _ in range(2)]
  for t in range(12):
    prev = xs[t]
    for l, (F, _, P) in enumerate(enc_dims):
      prev, _ = _gru(prev, states[l], *enc[l], F, P, ew)
      states[l] = prev
  layer_in = jnp.zeros((N, 1), jnp.float32)
  outs = []
  for t in range(12):
    prev = layer_in
    y8 = None
    for l, (F, _, P) in enumerate(dec_dims):
      dl_arg = (WdP, bdP) if l == 1 else None
      prev, y8 = _gru(prev, states[l], *dec[l], F, P, ew, dec_lin=dl_arg)
      states[l] = prev
    y = y8[:, :1]
    outs.append(y[:, 0])
    layer_in = y
  return jnp.stack(outs, axis=0)[None]


# speedup vs baseline: 1.4361x; 1.0255x over previous
import functools

import jax
import jax.numpy as jnp
from jax import lax
from jax.experimental import pallas as pl
from jax.experimental.pallas import tpu as pltpu
from jax.experimental.pallas import tpu_sc as plsc

N = 10000
N2 = 20000
E2 = 320000
NW = 32
NC = 2
RPW = 640
N2P = NW * RPW
CHK = 128
NJ = 4
SUP = NJ * CHK
LE = E2 + NW * SUP


def _make_prop(C, R):
  mesh = plsc.VectorSubcoreMesh(core_axis_name="c", subcore_axis_name="s",
                                num_cores=NC, num_subcores=16)

  @functools.partial(
      pl.kernel,
      out_type=jax.ShapeDtypeStruct((N2P * C,), jnp.float32),
      mesh=mesh,
      scratch_types=[
          pltpu.VMEM((SUP,), jnp.int32),
          pltpu.VMEM((SUP,), jnp.float32),
          pltpu.VMEM((SUP,), jnp.int32),
          pltpu.VMEM((2, CHK, C), jnp.float32),
          pltpu.VMEM((RPW * C,), jnp.float32),
          pltpu.VMEM((2 * NW * 16,), jnp.int32),
          pltpu.SemaphoreType.DMA,
          pltpu.SemaphoreType.DMA,
          pltpu.SemaphoreType.DMA,
          pltpu.SemaphoreType.DMA,
          pltpu.SemaphoreType.DMA,
      ],
  )
  def prop(tbl, idxh, wh, dlh, meta, out,
           idx_st, w_st, dl_st, rows, acc, meta_v, g0, g1, s0, s1, s2):
    wid = lax.axis_index("s") * NC + lax.axis_index("c")
    pltpu.sync_copy(meta, meta_v)
    soff = meta_v[pl.ds(wid * 16, 16)][0]
    nsc = meta_v[pl.ds((NW + wid) * 16, 16)][0]

    def zrow(r, _):
      for u in range(8):
        acc[pl.ds((r * 8 + u) * 16, 16)] = jnp.zeros((16,), jnp.float32)
      return 0
    lax.fori_loop(0, RPW * C // 128, zrow, 0)

    gsem = [g0, g1]

    def sup_body(s, _):
      base = pl.multiple_of(soff + s * SUP, SUP)
      d1 = pltpu.async_copy(idxh.at[pl.ds(base, SUP)], idx_st, s0)
      d2 = pltpu.async_copy(wh.at[pl.ds(base, SUP)], w_st, s1)
      d3 = pltpu.async_copy(dlh.at[pl.ds(base, SUP)], dl_st, s2)
      d1.wait(); d2.wait(); d3.wait()
      descs = [None, None]
      descs[0] = pltpu.async_copy(tbl.at[idx_st.at[pl.ds(0, CHK)]],
                                  rows.at[0], g0)
      for j in range(NJ):
        b = j % 2
        descs[b].wait()
        if j < NJ - 1:
          nb = (j + 1) % 2
          descs[nb] = pltpu.async_copy(
              tbl.at[idx_st.at[pl.ds((j + 1) * CHK, CHK)]], rows.at[nb],
              gsem[nb])

        def blk_body(blk, _, j=j, b=b):
          e0 = j * CHK + blk * 16
          dlv = dl_st[pl.ds(e0, 16)]
          wv = w_st[pl.ds(e0, 16)]
          for e in range(16):
            dl_s = dlv[e]
            w_s = wv[e]
            rbase = dl_s * C
            for cc in range(C // 16):
              c = (cc + e) % (C // 16)
              vals = rows[b, blk * 16 + e, pl.ds(c * 16, 16)] * w_s
              plsc.addupdate(acc.at[pl.ds(rbase + c * 16, 16)], vals)
          return 0
        lax.fori_loop(0, CHK // 16, blk_body, 0)
      return 0
    lax.fori_loop(0, nsc, sup_body, 0)
    pltpu.sync_copy(acc, out.at[pl.ds(wid * (RPW * C), RPW * C)])

  return prop


_PROP_CACHE = {}


def _get_prop(C, R):
  if (C, R) not in _PROP_CACHE:
    _PROP_CACHE[(C, R)] = jax.jit(_make_prop(C, R))
  return _PROP_CACHE[(C, R)]


def _preprocess(edge_index, edge_weight):
  src, dst = edge_index[0], edge_index[1]
  idx1 = jnp.concatenate([src, dst])
  idx2 = jnp.concatenate([src, dst + N])
  dstf = jnp.concatenate([dst, src + N])
  srcf = idx2
  ef = jnp.concatenate([edge_weight, edge_weight])
  deg = jax.ops.segment_sum(ef, dstf, num_segments=N2)
  w = ef * (1.0 / jnp.clip(deg, 1.0))[srcf]
  order = jnp.argsort(dstf)
  ds = dstf[order]
  wid_e = ds // RPW
  dl = ds - wid_e * RPW
  cnt = jnp.bincount(wid_e, length=NW).astype(jnp.int32)
  padded = ((cnt + SUP - 1) // SUP) * SUP
  off = jnp.concatenate([jnp.zeros((1,), jnp.int32), jnp.cumsum(padded)[:-1]])
  prefix = jnp.concatenate([jnp.zeros((1,), jnp.int32), jnp.cumsum(cnt)[:-1]])
  pos = off[wid_e] + jnp.arange(E2, dtype=jnp.int32) - prefix[wid_e]
  i1L = jnp.zeros((LE,), jnp.int32).at[pos].set(idx1[order])
  i2L = jnp.zeros((LE,), jnp.int32).at[pos].set(idx2[order])
  wL = jnp.zeros((LE,), jnp.float32).at[pos].set(w[order])
  dlL = jnp.zeros((LE,), jnp.int32).at[pos].set(dl)
  meta = jnp.concatenate([jnp.repeat(off.astype(jnp.int32), 16),
                          jnp.repeat((padded // SUP).astype(jnp.int32), 16)])
  return dict(i1=i1L, i2=i2L, w=wL, dl=dlL, meta=meta)


BR = 400


def _make_conv_tc(P, F, kind):
  GB = N // BR
  fout = 128 if kind == "zr" else 64
  nspec = pl.BlockSpec((BR, P), lambda i: (i, 0))
  aspec = pl.BlockSpec((BR, P), lambda i: (i, 0))
  bspec = pl.BlockSpec((BR, P), lambda i: (i + GB, 0))
  wspec = pl.BlockSpec((5, P, fout), lambda i: (0, 0, 0))
  bbspec = pl.BlockSpec((1, fout), lambda i: (0, 0))
  sspec = pl.BlockSpec((BR, 64), lambda i: (i, 0))

  def acc5(inp, x1a, x1b, x2a, x2b, W, b):
    m0 = inp[...]
    return (m0 @ W[0] + x1a[...] @ W[1] + x1b[...] @ W[2]
            + (2.0 * x2a[...] - m0) @ W[3] + (2.0 * x2b[...] - m0) @ W[4]
            + b[...])

  if kind == "zr":
    def body(inp, x1a, x1b, x2a, x2b, W, b, o_inp2, o_u):
      zr = jax.nn.sigmoid(acc5(inp, x1a, x1b, x2a, x2b, W, b))
      r, u = zr[:, :64], zr[:, 64:]
      m0 = inp[...]
      pieces = [m0[:, :F], r * m0[:, F:F + 64]]
      if P - F - 64:
        pieces.append(jnp.zeros((BR, P - F - 64), jnp.float32))
      o_inp2[...] = jnp.concatenate(pieces, axis=1)
      o_u[...] = u
    return pl.pallas_call(
        body, grid=(GB,),
        in_specs=[nspec, aspec, bspec, aspec, bspec, wspec, bbspec],
        out_specs=[nspec, sspec],
        out_shape=[jax.ShapeDtypeStruct((N, P), jnp.float32),
                   jax.ShapeDtypeStruct((N, 64), jnp.float32)])

  if kind == "h":
    def body(inp2, x1a, x1b, x2a, x2b, W, b, u, st, o_h):
      c = jnp.tanh(acc5(inp2, x1a, x1b, x2a, x2b, W, b))
      o_h[...] = u[...] * st[...] + (1.0 - u[...]) * c
    return pl.pallas_call(
        body, grid=(GB,),
        in_specs=[nspec, aspec, bspec, aspec, bspec, wspec, bbspec,
                  sspec, sspec],
        out_specs=sspec,
        out_shape=jax.ShapeDtypeStruct((N, 64), jnp.float32))

  def body(inp2, x1a, x1b, x2a, x2b, W, b, u, st, Wd, bd, o_h, o_y):
    c = jnp.tanh(acc5(inp2, x1a, x1b, x2a, x2b, W, b))
    h = u[...] * st[...] + (1.0 - u[...]) * c
    o_h[...] = h
    o_y[...] = h @ Wd[...] + bd[...]
  return pl.pallas_call(
      body, grid=(GB,),
      in_specs=[nspec, aspec, bspec, aspec, bspec, wspec, bbspec,
                sspec, sspec,
                pl.BlockSpec((64, 8), lambda i: (0, 0)),
                pl.BlockSpec((1, 8), lambda i: (0, 0))],
      out_specs=[sspec, pl.BlockSpec((BR, 8), lambda i: (i, 0))],
      out_shape=[jax.ShapeDtypeStruct((N, 64), jnp.float32),
                 jax.ShapeDtypeStruct((N, 8), jnp.float32)])


_TC_CACHE = {}


def _get_tc(kind, P, F):
  if (kind, P, F) not in _TC_CACHE:
    _TC_CACHE[(kind, P, F)] = jax.jit(_make_conv_tc(P, F, kind))
  return _TC_CACHE[(kind, P, F)]


def _pad_w(W, cin, P, fout):
  Wr = W.reshape(5, cin, fout)
  return jnp.pad(Wr, ((0, 0), (0, P - cin), (0, 0)))


def _props(inp_p, C, ew):
  pk1 = _get_prop(C, N)
  pk2 = _get_prop(C, N2)
  x1 = pk1(inp_p, ew["i1"], ew["w"], ew["dl"], ew["meta"]).reshape(N2P, C)
  x2r = pk2(x1, ew["i2"], ew["w"], ew["dl"],
            ew["meta"]).reshape(N2P, C)
  return x1, x2r


def _gru(feat, stt, WzrP, bzr, WhP, bh, F, P, ew, dec_lin=None):
  pad = P - (F + 64)
  inp = jnp.pad(jnp.concatenate([feat, stt], axis=1), ((0, 0), (0, pad)))
  x1, x2 = _props(inp, P, ew)
  inp2, u = _get_tc("zr", P, F)(inp, x1, x1, x2, x2, WzrP, bzr)
  y1, y2 = _props(inp2, P, ew)
  if dec_lin is None:
    h = _get_tc("h", P, F)(inp2, y1, y1, y2, y2, WhP, bh, u, stt)
    return h, None
  h, y = _get_tc("hd", P, F)(inp2, y1, y1, y2, y2, WhP, bh, u, stt,
                             dec_lin[0], dec_lin[1])
  return h, y


def kernel(x, edge_index, edge_weight, enc0_zr_W, enc0_zr_b, enc0_h_W,
           enc0_h_b, enc1_zr_W, enc1_zr_b, enc1_h_W, enc1_h_b, dec0_zr_W,
           dec0_zr_b, dec0_h_W, dec0_h_b, dec1_zr_W, dec1_zr_b, dec1_h_W,
           dec1_h_b, dec_lin_W, dec_lin_b):
  ew = _preprocess(edge_index, edge_weight)
  enc_dims = [(2, 66, 128), (64, 128, 128)]
  dec_dims = [(1, 65, 128), (64, 128, 128)]
  enc = [(_pad_w(enc0_zr_W, 66, 128, 128), enc0_zr_b,
          _pad_w(enc0_h_W, 66, 128, 64), enc0_h_b),
         (_pad_w(enc1_zr_W, 128, 128, 128), enc1_zr_b,
          _pad_w(enc1_h_W, 128, 128, 64), enc1_h_b)]
  dec = [(_pad_w(dec0_zr_W, 65, 128, 128), dec0_zr_b,
          _pad_w(dec0_h_W, 65, 128, 64), dec0_h_b),
         (_pad_w(dec1_zr_W, 128, 128, 128), dec1_zr_b,
          _pad_w(dec1_h_W, 128, 128, 64), dec1_h_b)]

  enc = [tuple(w if w.ndim != 1 else w.reshape(1, -1) for w in ws)
         for ws in enc]
  dec = [tuple(w if w.ndim != 1 else w.reshape(1, -1) for w in ws)
         for ws in dec]
  WdP = jnp.pad(dec_lin_W, ((0, 0), (0, 7)))
  bdP = jnp.pad(dec_lin_b.reshape(1, 1), ((0, 0), (0, 7)))

  xs = x[0]
  states = [jnp.zeros((N, 64), jnp.float32) for _ in range(2)]
  for t in range(12):
    prev = xs[t]
    for l, (F, _, P) in enumerate(enc_dims):
      prev, _ = _gru(prev, states[l], *enc[l], F, P, ew)
      states[l] = prev
  layer_in = jnp.zeros((N, 1), jnp.float32)
  outs = []
  for t in range(12):
    prev = layer_in
    y8 = None
    for l, (F, _, P) in enumerate(dec_dims):
      dl_arg = (WdP, bdP) if l == 1 else None
      prev, y8 = _gru(prev, states[l], *dec[l], F, P, ew, dec_lin=dl_arg)
      states[l] = prev
    y = y8[:, :1]
    outs.append(y[:, 0])
    layer_in = y
  return jnp.stack(outs, axis=0)[None]
